# Initial kernel scaffold; baseline (speedup 1.0000x reference)
import jax
import jax.numpy as jnp
from jax.experimental import pallas as pl


def kernel(x, pos_table, pe):
    raise NotImplementedError("write your pallas kernel here")


# speedup vs baseline: 2.0733x; 2.0733x over previous
import jax
import jax.numpy as jnp
from jax.experimental import pallas as pl

_BS = 256


def _pe_kernel(x_ref, pos_ref, pe_ref, out_ref):
    combined = 0.7 * pos_ref[...] + 0.3 * pe_ref[...]
    out_ref[...] = x_ref[...] + combined[None, :, :]


def kernel(x, pos_table, pe):
    batch, seq_len, d_model = x.shape
    bs = min(_BS, seq_len)
    grid = (seq_len // bs,)
    return pl.pallas_call(
        _pe_kernel,
        grid=grid,
        in_specs=[
            pl.BlockSpec((batch, bs, d_model), lambda i: (0, i, 0)),
            pl.BlockSpec((bs, d_model), lambda i: (i, 0)),
            pl.BlockSpec((bs, d_model), lambda i: (i, 0)),
        ],
        out_specs=pl.BlockSpec((batch, bs, d_model), lambda i: (0, i, 0)),
        out_shape=jax.ShapeDtypeStruct(x.shape, x.dtype),
    )(x, pos_table, pe)


# speedup vs baseline: 2.0895x; 1.0078x over previous
import jax
import jax.numpy as jnp
from jax.experimental import pallas as pl

_BS = 512


def _pe_kernel(x_ref, pos_ref, pe_ref, out_ref):
    combined = 0.7 * pos_ref[...] + 0.3 * pe_ref[...]
    out_ref[...] = x_ref[...] + combined[None, :, :]


def kernel(x, pos_table, pe):
    batch, seq_len, d_model = x.shape
    bs = min(_BS, seq_len)
    grid = (seq_len // bs,)
    return pl.pallas_call(
        _pe_kernel,
        grid=grid,
        in_specs=[
            pl.BlockSpec((batch, bs, d_model), lambda i: (0, i, 0)),
            pl.BlockSpec((bs, d_model), lambda i: (i, 0)),
            pl.BlockSpec((bs, d_model), lambda i: (i, 0)),
        ],
        out_specs=pl.BlockSpec((batch, bs, d_model), lambda i: (0, i, 0)),
        out_shape=jax.ShapeDtypeStruct(x.shape, x.dtype),
    )(x, pos_table, pe)
